# baseline (speedup 1.0000x reference)
import functools

import jax
import jax.numpy as jnp
from jax import lax
from jax.experimental import pallas as pl
from jax.experimental.pallas import tpu as pltpu
from jax.experimental.pallas import tpu_sc as plsc

D = 32
MU = 3.5
L = 16


@functools.cache
def _build(B: int):
    info = plsc.get_sparse_core_info()
    NC, NS = info.num_cores, info.num_subcores
    NW = NC * NS
    assert B % (8 * NW) == 0
    bpw = B // NW
    groups = bpw // L

    mesh = plsc.VectorSubcoreMesh(core_axis_name="c", subcore_axis_name="s")

    @functools.partial(
        pl.kernel,
        out_type=jax.ShapeDtypeStruct((B,), jnp.float32),
        mesh=mesh,
        compiler_params=pltpu.CompilerParams(
            needs_layout_passes=False, use_tc_tiling_on_sc=False),
        scratch_types=[
            pltpu.VMEM((bpw,), jnp.int32),
            pltpu.VMEM((bpw,), jnp.int32),
            pltpu.VMEM((bpw, D), jnp.float32),
            pltpu.VMEM((bpw, D), jnp.float32),
            pltpu.VMEM((bpw,), jnp.float32),
            pltpu.VMEM((bpw,), jnp.float32),
            pltpu.VMEM((bpw,), jnp.float32),
            pltpu.SemaphoreType.DMA,
        ],
    )
    def k(uidx_hbm, iidx_hbm, eu_hbm, ei_hbm, ub_hbm, ib_hbm, out_hbm,
          uidx_v, iidx_v, urows_v, irows_v, ubias_v, ibias_v, out_v, sem):
        wid = lax.axis_index("s") * NC + lax.axis_index("c")
        base = wid * bpw
        pltpu.sync_copy(uidx_hbm.at[pl.ds(base, bpw)], uidx_v)
        pltpu.sync_copy(iidx_hbm.at[pl.ds(base, bpw)], iidx_v)
        c0 = pltpu.async_copy(eu_hbm.at[uidx_v], urows_v, sem)
        c1 = pltpu.async_copy(ei_hbm.at[iidx_v], irows_v, sem)
        c2 = pltpu.async_copy(ub_hbm.at[uidx_v], ubias_v, sem)
        c3 = pltpu.async_copy(ib_hbm.at[iidx_v], ibias_v, sem)
        c0.wait()
        c1.wait()
        c2.wait()
        c3.wait()

        iota = lax.iota(jnp.int32, L)

        def body(g, carry):
            rows = g * L + iota
            acc = ubias_v[pl.ds(g * L, L)] + ibias_v[pl.ds(g * L, L)] + MU
            for d in range(D):
                cols = jnp.full((L,), d, jnp.int32)
                u = plsc.load_gather(urows_v, [rows, cols])
                v = plsc.load_gather(irows_v, [rows, cols])
                acc = acc + u * v
            out_v[pl.ds(g * L, L)] = acc
            return carry

        lax.fori_loop(0, groups, body, 0)
        pltpu.sync_copy(out_v, out_hbm.at[pl.ds(base, bpw)])

    return k


def kernel(user_idx, item_idx, embed_user, embed_item, user_bias, item_bias):
    B = user_idx.shape[0]
    k = _build(B)
    return k(user_idx.astype(jnp.int32), item_idx.astype(jnp.int32),
             embed_user, embed_item,
             jnp.squeeze(user_bias, axis=1), jnp.squeeze(item_bias, axis=1))
